# Initial kernel scaffold; baseline (speedup 1.0000x reference)
import jax
import jax.numpy as jnp
from jax.experimental import pallas as pl


def kernel(x_movement, x_phase, x_intersection, edge_index_mp, edge_index_pp, edge_index_pi, edge_attr_mp, edge_attr_pp, edge_attr_pi, params):
    raise NotImplementedError("write your pallas kernel here")


# speedup vs baseline: 26.5566x; 26.5566x over previous
import functools

import numpy as np
import jax
import jax.numpy as jnp
from jax import lax
from jax._src import config as _jax_config
from jax.experimental import pallas as pl
from jax.experimental.pallas import tpu as pltpu
from jax.experimental.pallas import tpu_sc as plsc

HID = 128
HEADS = 8
DH = HID // HEADS
DTAB = 16 + HID
KCH = 128
NTILES = 32

_G_np = np.zeros((HID, HEADS), np.float32)
for _h in range(HEADS):
    _G_np[_h * DH:(_h + 1) * DH, _h] = 1.0


def _prep_src_body(x_ref, wbig_ref, tab_ref, mx_ref):
    tab = jnp.dot(x_ref[...], wbig_ref[...], preferred_element_type=jnp.float32)
    tab_ref[...] = tab
    m = jnp.max(tab[:, 0:8], axis=0, keepdims=True)
    m = jnp.concatenate([m, jnp.full((1, 120), -1e30, jnp.float32)], axis=1)

    @pl.when(pl.program_id(0) == 0)
    def _():
        mx_ref[...] = m

    @pl.when(pl.program_id(0) != 0)
    def _():
        mx_ref[...] = jnp.maximum(mx_ref[...], m)


def _prep_src(x, wbig, blk):
    n, d = x.shape
    return pl.pallas_call(
        _prep_src_body,
        grid=(n // blk,),
        in_specs=[
            pl.BlockSpec((blk, d), lambda i: (i, 0)),
            pl.BlockSpec((d, DTAB), lambda i: (0, 0)),
        ],
        out_specs=[
            pl.BlockSpec((blk, DTAB), lambda i: (i, 0)),
            pl.BlockSpec((1, 128), lambda i: (0, 0)),
        ],
        out_shape=[
            jax.ShapeDtypeStruct((n, DTAB), jnp.float32),
            jax.ShapeDtypeStruct((1, 128), jnp.float32),
        ],
    )(x, wbig)


def _prep_dst_body(x_ref, adst_ref, wskip_ref, tab_ref, skip_ref, mx_ref):
    x = x_ref[...]
    t = jnp.dot(x, adst_ref[...], preferred_element_type=jnp.float32)
    tab_ref[...] = t
    skip_ref[...] = jnp.dot(x, wskip_ref[...], preferred_element_type=jnp.float32)
    m = jnp.max(t[:, 0:8], axis=0, keepdims=True)
    m = jnp.concatenate([m, jnp.full((1, 120), -1e30, jnp.float32)], axis=1)

    @pl.when(pl.program_id(0) == 0)
    def _():
        mx_ref[...] = m

    @pl.when(pl.program_id(0) != 0)
    def _():
        mx_ref[...] = jnp.maximum(mx_ref[...], m)


def _prep_dst(x, adst, wskip, blk):
    n, d = x.shape
    return pl.pallas_call(
        _prep_dst_body,
        grid=(n // blk,),
        in_specs=[
            pl.BlockSpec((blk, d), lambda i: (i, 0)),
            pl.BlockSpec((d, 16), lambda i: (0, 0)),
            pl.BlockSpec((d, HID), lambda i: (0, 0)),
        ],
        out_specs=[
            pl.BlockSpec((blk, 16), lambda i: (i, 0)),
            pl.BlockSpec((blk, HID), lambda i: (i, 0)),
            pl.BlockSpec((1, 128), lambda i: (0, 0)),
        ],
        out_shape=[
            jax.ShapeDtypeStruct((n, 16), jnp.float32),
            jax.ShapeDtypeStruct((n, HID), jnp.float32),
            jax.ShapeDtypeStruct((1, 128), jnp.float32),
        ],
    )(x, adst, wskip)


def _prep_edge_body(x_ref, aedge_ref, tab_ref, mx_ref):
    t = jnp.dot(x_ref[...], aedge_ref[...], preferred_element_type=jnp.float32)
    tab_ref[...] = t
    m = jnp.max(t[:, 0:8], axis=0, keepdims=True)
    m = jnp.concatenate([m, jnp.full((1, 120), -1e30, jnp.float32)], axis=1)

    @pl.when(pl.program_id(0) == 0)
    def _():
        mx_ref[...] = m

    @pl.when(pl.program_id(0) != 0)
    def _():
        mx_ref[...] = jnp.maximum(mx_ref[...], m)


def _prep_edge(ea, aedge, blk):
    n, d = ea.shape
    return pl.pallas_call(
        _prep_edge_body,
        grid=(n // blk,),
        in_specs=[
            pl.BlockSpec((blk, d), lambda i: (i, 0)),
            pl.BlockSpec((d, 16), lambda i: (0, 0)),
        ],
        out_specs=[
            pl.BlockSpec((blk, 16), lambda i: (i, 0)),
            pl.BlockSpec((1, 128), lambda i: (0, 0)),
        ],
        out_shape=[
            jax.ShapeDtypeStruct((n, 16), jnp.float32),
            jax.ShapeDtypeStruct((1, 128), jnp.float32),
        ],
    )(ea, aedge)


def _finish_body(acc_ref, gt_ref, wupd_ref, b_ref, skip_ref, out_ref):
    den = acc_ref[0, :, 0:8] + acc_ref[1, :, 0:8]
    num = acc_ref[0, :, 16:DTAB] + acc_ref[1, :, 16:DTAB]
    dene = jnp.dot(den, gt_ref[...], preferred_element_type=jnp.float32)
    agg = jnp.where(dene > 0.0, num / jnp.where(dene > 0.0, dene, 1.0), 0.0)
    y = jnp.dot(agg, wupd_ref[...], preferred_element_type=jnp.float32)
    out_ref[...] = jnp.maximum(y + b_ref[...] + skip_ref[...], 0.0)


def _finish(acc2, gt, wupd, b2d, skip, blk):
    n = skip.shape[0]
    return pl.pallas_call(
        _finish_body,
        grid=(n // blk,),
        in_specs=[
            pl.BlockSpec((2, blk, DTAB), lambda i: (0, i, 0)),
            pl.BlockSpec((HEADS, HID), lambda i: (0, 0)),
            pl.BlockSpec((HID, HID), lambda i: (0, 0)),
            pl.BlockSpec((1, HID), lambda i: (0, 0)),
            pl.BlockSpec((blk, HID), lambda i: (i, 0)),
        ],
        out_specs=pl.BlockSpec((blk, HID), lambda i: (i, 0)),
        out_shape=jax.ShapeDtypeStruct((n, HID), jnp.float32),
    )(acc2, gt, wupd, b2d, skip)


@functools.cache
def _make_sc_kernel(e_pad, n_pad):
    nchunks = e_pad // NTILES // KCH
    rpt = n_pad // 16
    nz = rpt // KCH
    assert rpt % KCH == 0

    mesh = plsc.VectorSubcoreMesh(core_axis_name="c", subcore_axis_name="s")

    @functools.partial(
        pl.kernel,
        out_type=jax.ShapeDtypeStruct((2, n_pad, DTAB), jnp.float32),
        mesh=mesh,
        compiler_params=pltpu.CompilerParams(
            needs_layout_passes=False, use_tc_tiling_on_sc=False),
        scratch_types=[
            pltpu.VMEM((KCH, DTAB), jnp.float32),
            pltpu.VMEM((KCH, 16), jnp.float32),
            pltpu.VMEM((KCH, 16), jnp.float32),
            pltpu.VMEM((KCH,), jnp.int32),
            pltpu.VMEM((KCH,), jnp.int32),
            pltpu.VMEM((32,), jnp.float32),
            pltpu.VMEM((16,), jnp.float32),
            pltpu.VMEM_SHARED((n_pad, DTAB), jnp.float32),
            pltpu.SemaphoreType.DMA,
            pltpu.SemaphoreType.DMA,
            pltpu.SemaphoreType.DMA,
        ],
    )
    def sck(src_tab, dst_tab, edge_tab, sidx2, didx2, cvec,
            out,
            srcbuf, dstbuf, edgebuf, sidx, didx, exbuf, cbuf, acc,
            sem1, sem2, sem3):
        c = lax.axis_index("c").astype(jnp.int32)
        s = lax.axis_index("s").astype(jnp.int32)
        w = c * jnp.int32(16) + s

        pltpu.sync_copy(cvec, cbuf)
        cv = cbuf[...]

        zeros16 = jnp.zeros((16,), jnp.float32)

        @pl.loop(0, KCH)
        def _zrow(i):
            for j in range(DTAB // 16):
                srcbuf[i, pl.ds(j * 16, 16)] = zeros16

        r0 = s * jnp.int32(rpt)
        for z in range(nz):
            pltpu.sync_copy(srcbuf, acc.at[pl.ds(r0 + z * KCH, KCH), :])
        plsc.subcore_barrier()

        idx_h = [jnp.full((16,), 16 + h, jnp.int32) for h in range(HEADS)]

        @pl.loop(0, nchunks)
        def _chunk_body(g):
            ci = w * jnp.int32(nchunks) + g
            pltpu.sync_copy(sidx2.at[ci], sidx)
            pltpu.sync_copy(didx2.at[ci], didx)
            cp1 = pltpu.async_copy(src_tab.at[sidx], srcbuf, sem1)
            cp2 = pltpu.async_copy(dst_tab.at[didx], dstbuf, sem2)
            cp3 = pltpu.async_copy(edge_tab.at[pl.ds(ci * jnp.int32(KCH), KCH), :],
                                   edgebuf, sem3)
            cp1.wait()
            cp2.wait()
            cp3.wait()

            @pl.loop(0, KCH)
            def _ebody(e):
                vs = srcbuf[e, pl.ds(0, 16)]
                vd = dstbuf[e, :]
                ve = edgebuf[e, :]
                l = vs + vd + ve
                l = jnp.where(l > 0.0, l, l * 0.2)
                exv = jnp.exp(l - cv)
                srcbuf[e, pl.ds(0, 16)] = exv
                exbuf[pl.ds(16, 16)] = exv
                for h in range(HEADS):
                    sl = pl.ds(16 + h * 16, 16)
                    scale = plsc.load_gather(exbuf, [idx_h[h]])
                    srcbuf[e, sl] = srcbuf[e, sl] * scale

            pltpu.sync_copy(srcbuf, acc.at[didx], add=True)

        plsc.subcore_barrier()

        for z in range(nz):
            rr = pl.ds(r0 + z * KCH, KCH)
            pltpu.sync_copy(acc.at[rr, :], srcbuf)
            pltpu.sync_copy(srcbuf, out.at[c, rr, :])

    return sck


def _layer(x_src, x_dst, edge_index, edge_attr, p, n_dst):
    G = jnp.asarray(_G_np)
    n_src, d_src = x_src.shape
    d_dst = x_dst.shape[1]
    E = edge_attr.shape[0]
    d_edge = edge_attr.shape[1]

    a_src_f = p['a_src'].reshape(1, HID)
    a_dst_f = p['a_dst'].reshape(1, HID)
    a_edge_f = p['a_edge'].reshape(1, HID)
    A_src = (p['W_src'] * a_src_f) @ G
    A_dst = (p['W_dst'] * a_dst_f) @ G
    A_edge = (p['W_edge'] * a_edge_f) @ G
    wbig = jnp.concatenate(
        [A_src, jnp.zeros((d_src, 8), jnp.float32), p['W_src']], axis=1)
    adst16 = jnp.concatenate([A_dst, jnp.zeros((d_dst, 8), jnp.float32)], axis=1)
    aedge16 = jnp.concatenate([A_edge, jnp.zeros((d_edge, 8), jnp.float32)], axis=1)

    e_pad = ((E + NTILES * KCH - 1) // (NTILES * KCH)) * (NTILES * KCH)
    ea_p = jnp.zeros((e_pad, d_edge), jnp.float32).at[:E, :].set(edge_attr)
    sidx = jnp.zeros((e_pad,), jnp.int32).at[:E].set(edge_index[0].astype(jnp.int32))
    didx = jnp.full((e_pad,), n_dst, jnp.int32).at[:E].set(edge_index[1].astype(jnp.int32))
    sidx2 = sidx.reshape(-1, KCH)
    didx2 = didx.reshape(-1, KCH)

    src_blk = 2000
    dst_blk = 2000

    src_tab, msrc = _prep_src(x_src, wbig, src_blk)
    dst_tab, skip, mdst = _prep_dst(x_dst, adst16, p['W_skip'], dst_blk)
    edge_tab, medge = _prep_edge(ea_p, aedge16, 4096)

    cm = msrc[0, 0:8] + mdst[0, 0:8] + medge[0, 0:8]
    c8 = jnp.where(cm > 0.0, cm, 0.2 * cm)
    cvec = jnp.concatenate([c8, jnp.zeros((8,), jnp.float32)]).astype(jnp.float32)

    dst_tab_p = jnp.concatenate(
        [dst_tab, jnp.zeros((1, 16), jnp.float32)], axis=0)

    n_pad = ((n_dst + 2047) // 2048) * 2048
    sck = _make_sc_kernel(e_pad, n_pad)
    acc2 = sck(src_tab, dst_tab_p, edge_tab, sidx2, didx2, cvec)

    b2d = p['b_upd'].reshape(1, HID).astype(jnp.float32)
    return _finish(acc2, jnp.asarray(_G_np.T), p['W_upd'], b2d, skip, dst_blk)


def kernel(x_movement, x_phase, x_intersection,
           edge_index_mp, edge_index_pp, edge_index_pi,
           edge_attr_mp, edge_attr_pp, edge_attr_pi, params):
    n_ph = x_phase.shape[0]
    n_int = x_intersection.shape[0]
    with _jax_config.enable_x64(False):
        ei_mp = edge_index_mp.astype(jnp.int32)
        ei_pp = edge_index_pp.astype(jnp.int32)
        ei_pi = edge_index_pi.astype(jnp.int32)
        xp = _layer(x_movement, x_phase, ei_mp, edge_attr_mp,
                    params['l1'], n_ph)
        xp = _layer(xp, xp, ei_pp, edge_attr_pp, params['l2'], n_ph)
        xi = _layer(xp, x_intersection, ei_pi, edge_attr_pi,
                    params['l3'], n_int)
    return (x_movement, xp, xi)


# speedup vs baseline: 32.7511x; 1.2333x over previous
import functools

import numpy as np
import jax
import jax.numpy as jnp
from jax import lax
from jax._src import config as _jax_config
from jax.experimental import pallas as pl
from jax.experimental.pallas import tpu as pltpu
from jax.experimental.pallas import tpu_sc as plsc

HID = 128
HEADS = 8
DH = HID // HEADS
DTAB = 16 + HID
KCH = 64
NTILES = 32

_G_np = np.zeros((HID, HEADS), np.float32)
for _h in range(HEADS):
    _G_np[_h * DH:(_h + 1) * DH, _h] = 1.0


def _prep_src_body(x_ref, wbig_ref, tab_ref, mx_ref):
    tab = jnp.dot(x_ref[...], wbig_ref[...], preferred_element_type=jnp.float32)
    tab_ref[...] = tab
    m = jnp.max(tab[:, 0:8], axis=0, keepdims=True)
    m = jnp.concatenate([m, jnp.full((1, 120), -1e30, jnp.float32)], axis=1)

    @pl.when(pl.program_id(0) == 0)
    def _():
        mx_ref[...] = m

    @pl.when(pl.program_id(0) != 0)
    def _():
        mx_ref[...] = jnp.maximum(mx_ref[...], m)


def _prep_src(x, wbig, blk):
    n, d = x.shape
    return pl.pallas_call(
        _prep_src_body,
        grid=(n // blk,),
        in_specs=[
            pl.BlockSpec((blk, d), lambda i: (i, 0)),
            pl.BlockSpec((d, DTAB), lambda i: (0, 0)),
        ],
        out_specs=[
            pl.BlockSpec((blk, DTAB), lambda i: (i, 0)),
            pl.BlockSpec((1, 128), lambda i: (0, 0)),
        ],
        out_shape=[
            jax.ShapeDtypeStruct((n, DTAB), jnp.float32),
            jax.ShapeDtypeStruct((1, 128), jnp.float32),
        ],
    )(x, wbig)


def _prep_dst_body(x_ref, adst_ref, wskip_ref, tab_ref, skip_ref, mx_ref):
    x = x_ref[...]
    t = jnp.dot(x, adst_ref[...], preferred_element_type=jnp.float32)
    tab_ref[...] = t
    skip_ref[...] = jnp.dot(x, wskip_ref[...], preferred_element_type=jnp.float32)
    m = jnp.max(t[:, 0:8], axis=0, keepdims=True)
    m = jnp.concatenate([m, jnp.full((1, 120), -1e30, jnp.float32)], axis=1)

    @pl.when(pl.program_id(0) == 0)
    def _():
        mx_ref[...] = m

    @pl.when(pl.program_id(0) != 0)
    def _():
        mx_ref[...] = jnp.maximum(mx_ref[...], m)


def _prep_dst(x, adst, wskip, blk):
    n, d = x.shape
    return pl.pallas_call(
        _prep_dst_body,
        grid=(n // blk,),
        in_specs=[
            pl.BlockSpec((blk, d), lambda i: (i, 0)),
            pl.BlockSpec((d, 16), lambda i: (0, 0)),
            pl.BlockSpec((d, HID), lambda i: (0, 0)),
        ],
        out_specs=[
            pl.BlockSpec((blk, 16), lambda i: (i, 0)),
            pl.BlockSpec((blk, HID), lambda i: (i, 0)),
            pl.BlockSpec((1, 128), lambda i: (0, 0)),
        ],
        out_shape=[
            jax.ShapeDtypeStruct((n, 16), jnp.float32),
            jax.ShapeDtypeStruct((n, HID), jnp.float32),
            jax.ShapeDtypeStruct((1, 128), jnp.float32),
        ],
    )(x, adst, wskip)


def _prep_edge_body(x_ref, aedge_ref, tab_ref, mx_ref):
    t = jnp.dot(x_ref[...], aedge_ref[...], preferred_element_type=jnp.float32)
    tab_ref[...] = t
    m = jnp.max(t[:, 0:8], axis=0, keepdims=True)
    m = jnp.concatenate([m, jnp.full((1, 120), -1e30, jnp.float32)], axis=1)

    @pl.when(pl.program_id(0) == 0)
    def _():
        mx_ref[...] = m

    @pl.when(pl.program_id(0) != 0)
    def _():
        mx_ref[...] = jnp.maximum(mx_ref[...], m)


def _prep_edge(ea, aedge, blk):
    n, d = ea.shape
    return pl.pallas_call(
        _prep_edge_body,
        grid=(n // blk,),
        in_specs=[
            pl.BlockSpec((blk, d), lambda i: (i, 0)),
            pl.BlockSpec((d, 16), lambda i: (0, 0)),
        ],
        out_specs=[
            pl.BlockSpec((blk, 16), lambda i: (i, 0)),
            pl.BlockSpec((1, 128), lambda i: (0, 0)),
        ],
        out_shape=[
            jax.ShapeDtypeStruct((n, 16), jnp.float32),
            jax.ShapeDtypeStruct((1, 128), jnp.float32),
        ],
    )(ea, aedge)


def _finish_body(acc_ref, gt_ref, wupd_ref, b_ref, skip_ref, out_ref):
    den = acc_ref[0, :, 0:8] + acc_ref[1, :, 0:8]
    num = acc_ref[0, :, 16:DTAB] + acc_ref[1, :, 16:DTAB]
    dene = jnp.dot(den, gt_ref[...], preferred_element_type=jnp.float32)
    agg = jnp.where(dene > 0.0, num / jnp.where(dene > 0.0, dene, 1.0), 0.0)
    y = jnp.dot(agg, wupd_ref[...], preferred_element_type=jnp.float32)
    out_ref[...] = jnp.maximum(y + b_ref[...] + skip_ref[...], 0.0)


def _finish(acc2, gt, wupd, b2d, skip, blk):
    n = skip.shape[0]
    return pl.pallas_call(
        _finish_body,
        grid=(n // blk,),
        in_specs=[
            pl.BlockSpec((2, blk, DTAB), lambda i: (0, i, 0)),
            pl.BlockSpec((HEADS, HID), lambda i: (0, 0)),
            pl.BlockSpec((HID, HID), lambda i: (0, 0)),
            pl.BlockSpec((1, HID), lambda i: (0, 0)),
            pl.BlockSpec((blk, HID), lambda i: (i, 0)),
        ],
        out_specs=pl.BlockSpec((blk, HID), lambda i: (i, 0)),
        out_shape=jax.ShapeDtypeStruct((n, HID), jnp.float32),
    )(acc2, gt, wupd, b2d, skip)


@functools.cache
def _make_sc_kernel(e_pad, n_pad):
    nchunks = e_pad // NTILES // KCH
    rpt = n_pad // 16
    nz = rpt // KCH
    assert rpt % KCH == 0

    mesh = plsc.VectorSubcoreMesh(core_axis_name="c", subcore_axis_name="s")

    @functools.partial(
        pl.kernel,
        out_type=jax.ShapeDtypeStruct((2, n_pad, DTAB), jnp.float32),
        mesh=mesh,
        compiler_params=pltpu.CompilerParams(
            needs_layout_passes=False, use_tc_tiling_on_sc=False),
        scratch_types=[
            pltpu.VMEM((KCH, DTAB), jnp.float32),
            pltpu.VMEM((KCH, 16), jnp.float32),
            pltpu.VMEM((KCH, 16), jnp.float32),
            pltpu.VMEM((KCH,), jnp.int32),
            pltpu.VMEM((KCH,), jnp.int32),
            pltpu.VMEM((KCH, DTAB), jnp.float32),
            pltpu.VMEM((KCH, 16), jnp.float32),
            pltpu.VMEM((KCH, 16), jnp.float32),
            pltpu.VMEM((KCH,), jnp.int32),
            pltpu.VMEM((KCH,), jnp.int32),
            pltpu.VMEM((32,), jnp.float32),
            pltpu.VMEM((16,), jnp.float32),
            pltpu.VMEM_SHARED((n_pad, DTAB), jnp.float32),
            pltpu.SemaphoreType.DMA,
            pltpu.SemaphoreType.DMA,
        ],
    )
    def sck(src_tab, dst_tab, edge_tab, sidx2, didx2, cvec,
            out,
            srcbuf0, dstbuf0, edgebuf0, sidx0, didx0,
            srcbuf1, dstbuf1, edgebuf1, sidx1, didx1,
            exbuf, cbuf, acc, gsem0, gsem1):
        srcbuf = (srcbuf0, srcbuf1)
        dstbuf = (dstbuf0, dstbuf1)
        edgebuf = (edgebuf0, edgebuf1)
        sidx = (sidx0, sidx1)
        didx = (didx0, didx1)
        gsem = (gsem0, gsem1)
        c = lax.axis_index("c").astype(jnp.int32)
        s = lax.axis_index("s").astype(jnp.int32)
        w = c * jnp.int32(16) + s

        pltpu.sync_copy(cvec, cbuf)
        cv = cbuf[...]

        zeros16 = jnp.zeros((16,), jnp.float32)

        @pl.loop(0, KCH)
        def _zrow(i):
            for j in range(DTAB // 16):
                srcbuf[0][i, pl.ds(j * 16, 16)] = zeros16

        r0 = s * jnp.int32(rpt)
        for z in range(nz):
            pltpu.sync_copy(srcbuf[0], acc.at[pl.ds(r0 + z * KCH, KCH), :])
        plsc.subcore_barrier()

        idx_h = [jnp.full((16,), 16 + h, jnp.int32) for h in range(HEADS)]
        base = w * jnp.int32(nchunks)

        def issue(ci, b):
            pltpu.sync_copy(sidx2.at[ci], sidx[b])
            pltpu.sync_copy(didx2.at[ci], didx[b])
            pltpu.async_copy(src_tab.at[sidx[b]], srcbuf[b], gsem[b])
            pltpu.async_copy(dst_tab.at[didx[b]], dstbuf[b], gsem[b])
            pltpu.async_copy(edge_tab.at[pl.ds(ci * jnp.int32(KCH), KCH), :],
                             edgebuf[b], gsem[b])

        def wait_gathers(b):
            pltpu.make_async_copy(src_tab.at[sidx[b]], srcbuf[b], gsem[b]).wait()
            pltpu.make_async_copy(dst_tab.at[didx[b]], dstbuf[b], gsem[b]).wait()
            pltpu.make_async_copy(edge_tab.at[pl.ds(0, KCH), :],
                                  edgebuf[b], gsem[b]).wait()

        issue(base, 0)
        issue(base + jnp.int32(1), 1)

        @pl.loop(0, nchunks, step=2)
        def _chunk_pair(g):
            for b in range(2):
                ge = g + jnp.int32(b)
                wait_gathers(b)

                @pl.loop(0, KCH)
                def _ebody(e):
                    vs = srcbuf[b][e, pl.ds(0, 16)]
                    vd = dstbuf[b][e, :]
                    ve = edgebuf[b][e, :]
                    l = vs + vd + ve
                    l = jnp.where(l > 0.0, l, l * 0.2)
                    exv = jnp.exp(l - cv)
                    srcbuf[b][e, pl.ds(0, 16)] = exv
                    exbuf[pl.ds(16, 16)] = exv
                    for h in range(HEADS):
                        sl = pl.ds(16 + h * 16, 16)
                        scale = plsc.load_gather(exbuf, [idx_h[h]])
                        srcbuf[b][e, sl] = srcbuf[b][e, sl] * scale

                pltpu.sync_copy(srcbuf[b], acc.at[didx[b]], add=True)

                @pl.when(ge < jnp.int32(nchunks - 2))
                def _():
                    issue(base + ge + jnp.int32(2), b)

        plsc.subcore_barrier()

        for z in range(nz):
            rr = pl.ds(r0 + z * KCH, KCH)
            pltpu.sync_copy(acc.at[rr, :], srcbuf[0])
            pltpu.sync_copy(srcbuf[0], out.at[c, rr, :])

    return sck


def _layer(x_src, x_dst, edge_index, edge_attr, p, n_dst):
    G = jnp.asarray(_G_np)
    n_src, d_src = x_src.shape
    d_dst = x_dst.shape[1]
    E = edge_attr.shape[0]
    d_edge = edge_attr.shape[1]

    a_src_f = p['a_src'].reshape(1, HID)
    a_dst_f = p['a_dst'].reshape(1, HID)
    a_edge_f = p['a_edge'].reshape(1, HID)
    A_src = (p['W_src'] * a_src_f) @ G
    A_dst = (p['W_dst'] * a_dst_f) @ G
    A_edge = (p['W_edge'] * a_edge_f) @ G
    wbig = jnp.concatenate(
        [A_src, jnp.zeros((d_src, 8), jnp.float32), p['W_src']], axis=1)
    adst16 = jnp.concatenate([A_dst, jnp.zeros((d_dst, 8), jnp.float32)], axis=1)
    aedge16 = jnp.concatenate([A_edge, jnp.zeros((d_edge, 8), jnp.float32)], axis=1)

    e_pad = ((E + 2 * NTILES * KCH - 1) // (2 * NTILES * KCH)) * (2 * NTILES * KCH)
    ea_p = jnp.zeros((e_pad, d_edge), jnp.float32).at[:E, :].set(edge_attr)
    sidx = jnp.zeros((e_pad,), jnp.int32).at[:E].set(edge_index[0].astype(jnp.int32))
    didx = jnp.full((e_pad,), n_dst, jnp.int32).at[:E].set(edge_index[1].astype(jnp.int32))
    sidx2 = sidx.reshape(-1, KCH)
    didx2 = didx.reshape(-1, KCH)

    src_blk = 2000
    dst_blk = 2000

    src_tab, msrc = _prep_src(x_src, wbig, src_blk)
    dst_tab, skip, mdst = _prep_dst(x_dst, adst16, p['W_skip'], dst_blk)
    edge_tab, medge = _prep_edge(ea_p, aedge16, 4096)

    cm = msrc[0, 0:8] + mdst[0, 0:8] + medge[0, 0:8]
    c8 = jnp.where(cm > 0.0, cm, 0.2 * cm)
    cvec = jnp.concatenate([c8, jnp.zeros((8,), jnp.float32)]).astype(jnp.float32)

    dst_tab_p = jnp.concatenate(
        [dst_tab, jnp.zeros((1, 16), jnp.float32)], axis=0)

    n_pad = ((n_dst + 2047) // 2048) * 2048
    sck = _make_sc_kernel(e_pad, n_pad)
    acc2 = sck(src_tab, dst_tab_p, edge_tab, sidx2, didx2, cvec)

    b2d = p['b_upd'].reshape(1, HID).astype(jnp.float32)
    return _finish(acc2, jnp.asarray(_G_np.T), p['W_upd'], b2d, skip, dst_blk)


def kernel(x_movement, x_phase, x_intersection,
           edge_index_mp, edge_index_pp, edge_index_pi,
           edge_attr_mp, edge_attr_pp, edge_attr_pi, params):
    n_ph = x_phase.shape[0]
    n_int = x_intersection.shape[0]
    with _jax_config.enable_x64(False):
        ei_mp = edge_index_mp.astype(jnp.int32)
        ei_pp = edge_index_pp.astype(jnp.int32)
        ei_pi = edge_index_pi.astype(jnp.int32)
        xp = _layer(x_movement, x_phase, ei_mp, edge_attr_mp,
                    params['l1'], n_ph)
        xp = _layer(xp, xp, ei_pp, edge_attr_pp, params['l2'], n_ph)
        xi = _layer(xp, x_intersection, ei_pi, edge_attr_pi,
                    params['l3'], n_int)
    return (x_movement, xp, xi)
